# baseline (speedup 1.0000x reference)
import jax
import jax.numpy as jnp
from jax import lax
from jax.experimental import pallas as pl
from jax.experimental.pallas import tpu as pltpu

_TAU = 0.07
_Q, _K, _D = 2048, 16384, 256
_BQ = 128
_BK = 2048
_NQ = _Q // _BQ
_NK = _K // _BK


def _fused_body(y_ref, x_ref, o_ref, m_ref, s_ref):
    y = y_ref[...].astype(jnp.bfloat16)
    for k in range(_NK):
        x = x_ref[pl.ds(k * _BK, _BK), :]
        dot = lax.dot_general(
            y, x.astype(jnp.bfloat16), (((1,), (1,)), ((), ())),
            preferred_element_type=jnp.float32,
        )
        sqx = lax.dot_general(
            jnp.ones((1, _D), jnp.float32), x * x, (((1,), (1,)), ((), ())),
            preferred_element_type=jnp.float32,
            precision=lax.Precision.HIGHEST,
        )
        logits = (2.0 * dot - sqx) * (1.0 / _TAU)

        m_c = jnp.max(logits, axis=1, keepdims=True)
        e = jnp.exp(logits - m_c)
        s_c = jnp.sum(e, axis=1, keepdims=True)

        o_ref[:, pl.ds(k * _BK, _BK)] = e
        m_ref[k] = jnp.broadcast_to(m_c, (_BQ, 128))
        s_ref[k] = jnp.broadcast_to(s_c, (_BQ, 128))

    m_all = m_ref[...]
    s_all = s_ref[...]
    m = jnp.max(m_all, axis=0)
    w = jnp.exp(m_all - m)
    s = jnp.sum(s_all * w, axis=0)
    r = w / s
    for c in range(_NK):
        sl = pl.ds(c * _BK, _BK)
        o_ref[:, sl] = o_ref[:, sl] * r[c, :, 0:1]


def kernel(X, Y):
    return pl.pallas_call(
        _fused_body,
        grid=(_NQ,),
        in_specs=[
            pl.BlockSpec((_BQ, _D), lambda q: (q, 0)),
            pl.BlockSpec((_K, _D), lambda q: (0, 0)),
        ],
        out_specs=pl.BlockSpec((_BQ, _K), lambda q: (q, 0)),
        out_shape=jax.ShapeDtypeStruct((_Q, _K), jnp.float32),
        scratch_shapes=[
            pltpu.VMEM((_NK, _BQ, 128), jnp.float32),
            pltpu.VMEM((_NK, _BQ, 128), jnp.float32),
        ],
        compiler_params=pltpu.CompilerParams(
            dimension_semantics=("arbitrary",),
        ),
    )(Y, X)


# speedup vs baseline: 1.9661x; 1.9661x over previous
import jax
import jax.numpy as jnp
from jax import lax
from jax.experimental import pallas as pl
from jax.experimental.pallas import tpu as pltpu

_TAU = 0.07
_Q, _K, _D = 2048, 16384, 256
_BQ = 256
_BK = 2048
_NQ = _Q // _BQ
_NK = _K // _BK


def _bias_body(x_ref, b_ref):
    xx = x_ref[...]
    sqx = lax.dot_general(
        jnp.ones((1, _D), jnp.float32), xx * xx, (((1,), (1,)), ((), ())),
        preferred_element_type=jnp.float32,
        precision=lax.Precision.HIGHEST,
    )
    b_ref[...] = sqx * (-1.0 / _TAU)


def _fused_body(y_ref, x_ref, b_ref, o_ref, m_ref, s_ref):
    k = pl.program_id(1)
    dot = lax.dot_general(
        y_ref[...], x_ref[...], (((1,), (1,)), ((), ())),
        preferred_element_type=jnp.float32,
    )
    logits = dot * (2.0 / _TAU) + b_ref[...]

    m_c = jnp.max(logits, axis=1, keepdims=True)
    e = jnp.exp(logits - m_c)
    s_c = jnp.sum(e, axis=1, keepdims=True)

    o_ref[:, pl.ds(k * _BK, _BK)] = e
    m_ref[k] = jnp.broadcast_to(m_c, (_BQ, 128))
    s_ref[k] = jnp.broadcast_to(s_c, (_BQ, 128))

    @pl.when(k == _NK - 1)
    def _finalize():
        m_all = m_ref[...]
        s_all = s_ref[...]
        m = jnp.max(m_all, axis=0)
        w = jnp.exp(m_all - m)
        s = jnp.sum(s_all * w, axis=0)
        r = w / s
        for c in range(_NK):
            sl = pl.ds(c * _BK, _BK)
            o_ref[:, sl] = o_ref[:, sl] * r[c, :, 0:1]


def kernel(X, Y):
    bias = pl.pallas_call(
        _bias_body,
        grid=(1,),
        in_specs=[pl.BlockSpec((_K, _D), lambda i: (0, 0))],
        out_specs=pl.BlockSpec((1, _K), lambda i: (0, 0)),
        out_shape=jax.ShapeDtypeStruct((1, _K), jnp.float32),
    )(X)
    Xb = X.astype(jnp.bfloat16)
    Yb = Y.astype(jnp.bfloat16)
    return pl.pallas_call(
        _fused_body,
        grid=(_NQ, _NK),
        in_specs=[
            pl.BlockSpec((_BQ, _D), lambda q, k: (q, 0)),
            pl.BlockSpec((_BK, _D), lambda q, k: (k, 0)),
            pl.BlockSpec((1, _BK), lambda q, k: (0, k)),
        ],
        out_specs=pl.BlockSpec((_BQ, _K), lambda q, k: (q, 0)),
        out_shape=jax.ShapeDtypeStruct((_Q, _K), jnp.float32),
        scratch_shapes=[
            pltpu.VMEM((_NK, _BQ, 128), jnp.float32),
            pltpu.VMEM((_NK, _BQ, 128), jnp.float32),
        ],
        compiler_params=pltpu.CompilerParams(
            dimension_semantics=("parallel", "arbitrary"),
        ),
    )(Yb, Xb, bias)
